# baseline (speedup 1.0000x reference)
import functools

import jax
import jax.numpy as jnp
from jax import lax
from jax.experimental import pallas as pl
from jax.experimental.pallas import tpu as pltpu
from jax.experimental.pallas import tpu_sc as plsc

N = 100000
E = 3200000
F = 16
SUB = 128
JSUB = 10
CHUNK = SUB * JSUB
NCH = E // CHUNK
NTILES = 32
RPT = N // 16
ZROWS = 625


def _agg_body(src_hbm, dst_hbm, table_hbm, zeros_hbm,
              out_hbm,
              idx_s, idx_d, rows,
              semi, semg, sems, acc):
    c = lax.axis_index("c")
    s = lax.axis_index("s")
    wid = s * 2 + c

    base = s * RPT
    pltpu.sync_copy(zeros_hbm.at[pl.ds(base, RPT)], acc.at[pl.ds(base, RPT)])
    plsc.subcore_barrier()

    lo = (wid * NCH) // NTILES
    hi = ((wid + 1) * NCH) // NTILES

    def fire_idx(chunk, b):
        pltpu.async_copy(src_hbm.at[chunk], idx_s.at[b], semi.at[b])
        pltpu.async_copy(dst_hbm.at[chunk], idx_d.at[b], semi.at[b])

    fire_idx(lo, 0)

    def chunk_body(chunk, _):
        b = lax.rem(chunk - lo, 2)
        pltpu.make_async_copy(src_hbm.at[chunk], idx_s.at[b],
                              semi.at[b]).wait()
        pltpu.make_async_copy(dst_hbm.at[chunk], idx_d.at[b],
                              semi.at[b]).wait()

        @pl.when(chunk + 1 < hi)
        def _pref():
            fire_idx(chunk + 1, 1 - b)

        gd = [pltpu.async_copy(table_hbm.at[idx_s.at[b].at[j]],
                               rows.at[j], semg.at[j])
              for j in range(JSUB)]

        sd = []
        for j in range(JSUB):
            gd[j].wait()
            sd.append(pltpu.async_copy(rows.at[j], acc.at[idx_d.at[b].at[j]],
                                       sems.at[j], add=True))
        for d in sd:
            d.wait()
        return _
    lax.fori_loop(lo, hi, chunk_body, None)
    plsc.subcore_barrier()

    pltpu.sync_copy(acc.at[pl.ds(base, RPT)],
                    out_hbm.at[c, pl.ds(base, RPT)])


def _agg(src, dst, table, zeros_nw, w):
    mesh = plsc.VectorSubcoreMesh(core_axis_name="c", subcore_axis_name="s")
    k = pl.kernel(
        _agg_body,
        out_type=jax.ShapeDtypeStruct((2, N, w), jnp.float32),
        mesh=mesh,
        compiler_params=pltpu.CompilerParams(use_tc_tiling_on_sc=False,
                                             needs_layout_passes=False),
        scratch_types=[
            pltpu.VMEM((2, JSUB, SUB), jnp.int32),
            pltpu.VMEM((2, JSUB, SUB), jnp.int32),
            pltpu.VMEM((JSUB, SUB, w), jnp.float32),
            pltpu.SemaphoreType.DMA((2,)),
            pltpu.SemaphoreType.DMA((JSUB,)),
            pltpu.SemaphoreType.DMA((JSUB,)),
            pltpu.VMEM_SHARED((N, w), jnp.float32),
        ],
    )
    return k(src, dst, table, zeros_nw)


NOUT = 68
OPAD = 80


def _agg68_body(src_hbm, dst_hbm, table_hbm, out_hbm,
                srcb, dstb, rowbuf, acc, semi):
    c = lax.axis_index("c")
    s = lax.axis_index("s")
    wid = s * 2 + c

    def zfill(i, _):
        acc[i] = jnp.zeros((F,), jnp.float32)
        return _
    lax.fori_loop(0, OPAD, zfill, None)

    lo = (wid * NCH) // NTILES
    hi = ((wid + 1) * NCH) // NTILES

    def fire_idx(chunk, b):
        pltpu.async_copy(src_hbm.at[chunk], srcb.at[b], semi.at[b])
        pltpu.async_copy(dst_hbm.at[chunk], dstb.at[b], semi.at[b])

    fire_idx(lo, 0)

    def chunk_body(chunk, _):
        b = lax.rem(chunk - lo, 2)
        pltpu.make_async_copy(src_hbm.at[chunk], srcb.at[b],
                              semi.at[b]).wait()
        pltpu.make_async_copy(dst_hbm.at[chunk], dstb.at[b],
                              semi.at[b]).wait()

        @pl.when(chunk + 1 < hi)
        def _pref():
            fire_idx(chunk + 1, 1 - b)

        def sub_body(j, __):
            dvs = [dstb[b, j, pl.ds(k * 16, 16)] for k in range(8)]
            mins = functools.reduce(jnp.minimum, dvs)

            @pl.when(plsc.all_reduce_population_count(mins < NOUT)[0] > 0)
            def _hit():
                for k in range(8):
                    @pl.when(plsc.all_reduce_population_count(
                        dvs[k] < NOUT)[0] > 0)
                    def _grp(k=k):
                        sv = srcb[b, j, pl.ds(k * 16, 16)]
                        for l in range(16):
                            @pl.when(dvs[k][l] < NOUT)
                            def _edge(l=l):
                                pltpu.sync_copy(table_hbm.at[sv[l]], rowbuf)
                                d = dvs[k][l]
                                acc[d] = acc[d] + rowbuf[...]
            return __
        lax.fori_loop(0, JSUB, sub_body, None)
        return _
    lax.fori_loop(lo, hi, chunk_body, None)

    pltpu.sync_copy(acc, out_hbm.at[wid])


def _agg68(src, dst, table):
    mesh = plsc.VectorSubcoreMesh(core_axis_name="c", subcore_axis_name="s")
    k = pl.kernel(
        _agg68_body,
        out_type=jax.ShapeDtypeStruct((NTILES, OPAD, F), jnp.float32),
        mesh=mesh,
        compiler_params=pltpu.CompilerParams(use_tc_tiling_on_sc=False,
                                             needs_layout_passes=False),
        scratch_types=[
            pltpu.VMEM((2, JSUB, SUB), jnp.int32),
            pltpu.VMEM((2, JSUB, SUB), jnp.int32),
            pltpu.VMEM((F,), jnp.float32),
            pltpu.VMEM((OPAD, F), jnp.float32),
            pltpu.SemaphoreType.DMA((2,)),
        ],
    )
    return k(src, dst, table)


BLK = 5000
GRID = N // BLK


def _prep_body(x_ref, w_ref, b_ref, o_ref):
    i = pl.program_id(0)
    rows = (jnp.float32(i * BLK)
            + lax.broadcasted_iota(jnp.int32, (BLK, 1), 0).astype(jnp.float32))
    vect = jnp.tanh(rows * w_ref[...] + b_ref[...])
    o_ref[...] = jnp.concatenate(
        [x_ref[...], vect,
         jnp.ones((BLK, 1), jnp.float32),
         jnp.zeros((BLK, F - 9), jnp.float32)], axis=1)


def _prep(x, pos_W, pos_b):
    return pl.pallas_call(
        _prep_body,
        grid=(GRID,),
        in_specs=[
            pl.BlockSpec((BLK, 3), lambda i: (i, 0)),
            pl.BlockSpec((1, 5), lambda i: (0, 0)),
            pl.BlockSpec((1, 5), lambda i: (0, 0)),
        ],
        out_specs=pl.BlockSpec((BLK, F), lambda i: (i, 0)),
        out_shape=jax.ShapeDtypeStruct((N, F), jnp.float32),
    )(x, pos_W.reshape(1, 5), pos_b.reshape(1, 5))


def _dense1_body(p, h0, wl, bl, wr, h1_o, rcn_o, m1_o):
    s8 = p[0, :, :8] + p[1, :, :8]
    cnt = p[0, :, 8:9] + p[1, :, 8:9]
    rcn = 1.0 / jnp.maximum(cnt, 1.0)
    m1 = s8 * rcn
    h1_o[...] = (jnp.dot(m1, wl[...], preferred_element_type=jnp.float32)
                 + bl[...]
                 + jnp.dot(h0[:, :8], wr[...],
                           preferred_element_type=jnp.float32))
    rcn_o[...] = rcn
    m1_o[...] = m1


def _dense1(p, h0ext, Wl, bl, Wr):
    return pl.pallas_call(
        _dense1_body,
        grid=(GRID,),
        in_specs=[
            pl.BlockSpec((2, BLK, F), lambda i: (0, i, 0)),
            pl.BlockSpec((BLK, F), lambda i: (i, 0)),
            pl.BlockSpec((8, F), lambda i: (0, 0)),
            pl.BlockSpec((1, F), lambda i: (0, 0)),
            pl.BlockSpec((8, F), lambda i: (0, 0)),
        ],
        out_specs=[
            pl.BlockSpec((BLK, F), lambda i: (i, 0)),
            pl.BlockSpec((BLK, 1), lambda i: (i, 0)),
            pl.BlockSpec((BLK, 8), lambda i: (i, 0)),
        ],
        out_shape=[
            jax.ShapeDtypeStruct((N, F), jnp.float32),
            jax.ShapeDtypeStruct((N, 1), jnp.float32),
            jax.ShapeDtypeStruct((N, 8), jnp.float32),
        ],
    )(p, h0ext, Wl.T, bl.reshape(1, F), Wr.T)


def _dense2_body(p2, p, rcn, h1, wl1, bl1v, wr1, wl2, bl2v, wr2, o_ref):
    am1 = p2[0] + p2[1]
    agg1 = p[0, :, :8] + p[1, :, :8]
    cnt = p[0, :, 8:9] + p[1, :, 8:9]
    ah1 = (jnp.dot(am1, wl1[...], preferred_element_type=jnp.float32)
           + cnt * bl1v[...]
           + jnp.dot(agg1, wr1[...], preferred_element_type=jnp.float32))
    agg2 = ah1 * rcn[...]
    o_ref[...] = (jnp.dot(agg2, wl2[...], preferred_element_type=jnp.float32)
                  + bl2v[...]
                  + jnp.dot(h1[...], wr2[...],
                            preferred_element_type=jnp.float32))


def _dense2(p2, p, rcn, h1, Wl1, bl1, Wr1, Wl2, bl2, Wr2):
    return pl.pallas_call(
        _dense2_body,
        grid=(GRID,),
        in_specs=[
            pl.BlockSpec((2, BLK, 8), lambda i: (0, i, 0)),
            pl.BlockSpec((2, BLK, F), lambda i: (0, i, 0)),
            pl.BlockSpec((BLK, 1), lambda i: (i, 0)),
            pl.BlockSpec((BLK, F), lambda i: (i, 0)),
            pl.BlockSpec((8, F), lambda i: (0, 0)),
            pl.BlockSpec((1, F), lambda i: (0, 0)),
            pl.BlockSpec((8, F), lambda i: (0, 0)),
            pl.BlockSpec((F, F), lambda i: (0, 0)),
            pl.BlockSpec((1, F), lambda i: (0, 0)),
            pl.BlockSpec((F, F), lambda i: (0, 0)),
        ],
        out_specs=pl.BlockSpec((BLK, F), lambda i: (i, 0)),
        out_shape=jax.ShapeDtypeStruct((N, F), jnp.float32),
    )(p2, p, rcn, h1, Wl1.T, bl1.reshape(1, F), Wr1.T,
      Wl2.T, bl2.reshape(1, F), Wr2.T)


def _dense3_body(p, rcn, h, wl, bl, wr, o_ref):
    agg = jnp.sum(p[...], axis=0) * rcn[...]
    o_ref[...] = (jnp.dot(agg, wl[...], preferred_element_type=jnp.float32)
                  + bl[...]
                  + jnp.dot(h[...], wr[...],
                            preferred_element_type=jnp.float32))


def _dense3(p, rcn, h, Wl, bl, Wr):
    return pl.pallas_call(
        _dense3_body,
        out_shape=jax.ShapeDtypeStruct((OPAD, 3), jnp.float32),
    )(p, rcn, h, Wl.T, bl.reshape(1, 3), Wr.T)


def kernel(x, edge_index, pos_W, pos_b,
           Wl1, bl1, Wr1, Wl2, bl2, Wr2, Wl3, bl3, Wr3):
    src = edge_index[0].reshape(NCH, JSUB, SUB)
    dst = edge_index[1].reshape(NCH, JSUB, SUB)
    zeros_nf = jnp.zeros((N, F), jnp.float32)
    zeros_n8 = jnp.zeros((N, 8), jnp.float32)

    h0ext = _prep(x, pos_W, pos_b)
    p = _agg(src, dst, h0ext, zeros_nf, F)
    h1, rcn, m1 = _dense1(p, h0ext, Wl1, bl1, Wr1)
    p2 = _agg(src, dst, m1, zeros_n8, 8)
    h2 = _dense2(p2, p, rcn, h1, Wl1, bl1, Wr1, Wl2, bl2, Wr2)
    p3 = _agg68(src, dst, h2)
    out = _dense3(p3, rcn[:OPAD], h2[:OPAD], Wl3, bl3, Wr3)
    return out[:NOUT]


# speedup vs baseline: 1.0431x; 1.0431x over previous
import functools

import jax
import jax.numpy as jnp
from jax import lax
from jax.experimental import pallas as pl
from jax.experimental.pallas import tpu as pltpu
from jax.experimental.pallas import tpu_sc as plsc

N = 100000
E = 3200000
F = 16
SUB = 128
JSUB = 10
CHUNK = SUB * JSUB
NCH = E // CHUNK
NTILES = 32
RPT = N // 16
ZROWS = 625


def _agg_body(src_hbm, dst_hbm, table_hbm, zeros_hbm,
              out_hbm,
              idx_s, idx_d, rows,
              semi, semg, sems, acc):
    c = lax.axis_index("c")
    s = lax.axis_index("s")
    wid = s * 2 + c

    base = s * RPT
    pltpu.sync_copy(zeros_hbm.at[pl.ds(base, RPT)], acc.at[pl.ds(base, RPT)])
    plsc.subcore_barrier()

    lo = (wid * NCH) // NTILES
    hi = ((wid + 1) * NCH) // NTILES

    def fire_idx(chunk, b):
        pltpu.async_copy(src_hbm.at[chunk], idx_s.at[b], semi.at[b])
        pltpu.async_copy(dst_hbm.at[chunk], idx_d.at[b], semi.at[b])

    fire_idx(lo, 0)

    def chunk_body(chunk, _):
        b = lax.rem(chunk - lo, 2)
        pltpu.make_async_copy(src_hbm.at[chunk], idx_s.at[b],
                              semi.at[b]).wait()
        pltpu.make_async_copy(dst_hbm.at[chunk], idx_d.at[b],
                              semi.at[b]).wait()

        @pl.when(chunk + 1 < hi)
        def _pref():
            fire_idx(chunk + 1, 1 - b)

        gd = [pltpu.async_copy(table_hbm.at[idx_s.at[b].at[j]],
                               rows.at[j], semg.at[j])
              for j in range(JSUB)]

        sd = []
        for j in range(JSUB):
            gd[j].wait()
            sd.append(pltpu.async_copy(rows.at[j], acc.at[idx_d.at[b].at[j]],
                                       sems.at[j], add=True))
        for d in sd:
            d.wait()
        return _
    lax.fori_loop(lo, hi, chunk_body, None)
    plsc.subcore_barrier()

    pltpu.sync_copy(acc.at[pl.ds(base, RPT)],
                    out_hbm.at[c, pl.ds(base, RPT)])


def _agg(src, dst, table, zeros_nw, w):
    mesh = plsc.VectorSubcoreMesh(core_axis_name="c", subcore_axis_name="s")
    k = pl.kernel(
        _agg_body,
        out_type=jax.ShapeDtypeStruct((2, N, w), jnp.float32),
        mesh=mesh,
        compiler_params=pltpu.CompilerParams(use_tc_tiling_on_sc=False,
                                             needs_layout_passes=False),
        scratch_types=[
            pltpu.VMEM((2, JSUB, SUB), jnp.int32),
            pltpu.VMEM((2, JSUB, SUB), jnp.int32),
            pltpu.VMEM((JSUB, SUB, w), jnp.float32),
            pltpu.SemaphoreType.DMA((2,)),
            pltpu.SemaphoreType.DMA((JSUB,)),
            pltpu.SemaphoreType.DMA((JSUB,)),
            pltpu.VMEM_SHARED((N, w), jnp.float32),
        ],
    )
    return k(src, dst, table, zeros_nw)


NOUT = 68
OPAD = 80


def _agg68_body(src_hbm, dst_hbm, table_hbm, out_hbm,
                srcb, dstb, rowbuf, acc, semi):
    c = lax.axis_index("c")
    s = lax.axis_index("s")
    wid = s * 2 + c

    def zfill(i, _):
        acc[i] = jnp.zeros((F,), jnp.float32)
        return _
    lax.fori_loop(0, OPAD, zfill, None)

    lo = (wid * NCH) // NTILES
    hi = ((wid + 1) * NCH) // NTILES

    def fire_idx(chunk, b):
        pltpu.async_copy(src_hbm.at[chunk], srcb.at[b], semi.at[b])
        pltpu.async_copy(dst_hbm.at[chunk], dstb.at[b], semi.at[b])

    fire_idx(lo, 0)

    def chunk_body(chunk, _):
        b = lax.rem(chunk - lo, 2)
        pltpu.make_async_copy(src_hbm.at[chunk], srcb.at[b],
                              semi.at[b]).wait()
        pltpu.make_async_copy(dst_hbm.at[chunk], dstb.at[b],
                              semi.at[b]).wait()

        @pl.when(chunk + 1 < hi)
        def _pref():
            fire_idx(chunk + 1, 1 - b)

        def sub_body(j, __):
            dvs = [dstb[b, j, pl.ds(k * 16, 16)] for k in range(8)]
            mins = functools.reduce(jnp.minimum, dvs)

            @pl.when(plsc.all_reduce_population_count(mins < NOUT)[0] > 0)
            def _hit():
                for k in range(8):
                    @pl.when(plsc.all_reduce_population_count(
                        dvs[k] < NOUT)[0] > 0)
                    def _grp(k=k):
                        sv = srcb[b, j, pl.ds(k * 16, 16)]
                        for l in range(16):
                            @pl.when(dvs[k][l] < NOUT)
                            def _edge(l=l):
                                pltpu.sync_copy(table_hbm.at[sv[l]], rowbuf)
                                d = dvs[k][l]
                                acc[d] = acc[d] + rowbuf[...]
            return __
        lax.fori_loop(0, JSUB, sub_body, None)
        return _
    lax.fori_loop(lo, hi, chunk_body, None)

    pltpu.sync_copy(acc, out_hbm.at[wid])


def _agg68(src, dst, table):
    mesh = plsc.VectorSubcoreMesh(core_axis_name="c", subcore_axis_name="s")
    k = pl.kernel(
        _agg68_body,
        out_type=jax.ShapeDtypeStruct((NTILES, OPAD, F), jnp.float32),
        mesh=mesh,
        compiler_params=pltpu.CompilerParams(use_tc_tiling_on_sc=False,
                                             needs_layout_passes=False),
        scratch_types=[
            pltpu.VMEM((2, JSUB, SUB), jnp.int32),
            pltpu.VMEM((2, JSUB, SUB), jnp.int32),
            pltpu.VMEM((F,), jnp.float32),
            pltpu.VMEM((OPAD, F), jnp.float32),
            pltpu.SemaphoreType.DMA((2,)),
        ],
    )
    return k(src, dst, table)


BLK = 5000
GRID = N // BLK


def _prep_body(x_ref, w_ref, b_ref, o_ref):
    i = pl.program_id(0)
    rows = (jnp.float32(i * BLK)
            + lax.broadcasted_iota(jnp.int32, (BLK, 1), 0).astype(jnp.float32))
    vect = jnp.tanh(rows * w_ref[...] + b_ref[...])
    o_ref[...] = jnp.concatenate(
        [x_ref[...], vect,
         jnp.ones((BLK, 1), jnp.float32),
         jnp.zeros((BLK, F - 9), jnp.float32)], axis=1)


def _prep(x, pos_W, pos_b):
    return pl.pallas_call(
        _prep_body,
        grid=(GRID,),
        in_specs=[
            pl.BlockSpec((BLK, 3), lambda i: (i, 0)),
            pl.BlockSpec((1, 5), lambda i: (0, 0)),
            pl.BlockSpec((1, 5), lambda i: (0, 0)),
        ],
        out_specs=pl.BlockSpec((BLK, F), lambda i: (i, 0)),
        out_shape=jax.ShapeDtypeStruct((N, F), jnp.float32),
    )(x, pos_W.reshape(1, 5), pos_b.reshape(1, 5))


def _dense1_body(p, h0, wl, bl, wr, h1_o, rcn_o):
    s8 = p[0, :, :8] + p[1, :, :8]
    cnt = p[0, :, 8:9] + p[1, :, 8:9]
    rcn = 1.0 / jnp.maximum(cnt, 1.0)
    h1_o[...] = (jnp.dot(s8 * rcn, wl[...],
                         preferred_element_type=jnp.float32)
                 + bl[...]
                 + jnp.dot(h0[:, :8], wr[...],
                           preferred_element_type=jnp.float32))
    rcn_o[...] = rcn


def _dense1(p, h0ext, Wl, bl, Wr):
    return pl.pallas_call(
        _dense1_body,
        grid=(GRID,),
        in_specs=[
            pl.BlockSpec((2, BLK, F), lambda i: (0, i, 0)),
            pl.BlockSpec((BLK, F), lambda i: (i, 0)),
            pl.BlockSpec((8, F), lambda i: (0, 0)),
            pl.BlockSpec((1, F), lambda i: (0, 0)),
            pl.BlockSpec((8, F), lambda i: (0, 0)),
        ],
        out_specs=[
            pl.BlockSpec((BLK, F), lambda i: (i, 0)),
            pl.BlockSpec((BLK, 1), lambda i: (i, 0)),
        ],
        out_shape=[
            jax.ShapeDtypeStruct((N, F), jnp.float32),
            jax.ShapeDtypeStruct((N, 1), jnp.float32),
        ],
    )(p, h0ext, Wl.T, bl.reshape(1, F), Wr.T)


def _dense2_body(p, rcn, h, wl, bl, wr, o_ref):
    agg = (p[0] + p[1]) * rcn[...]
    o_ref[...] = (jnp.dot(agg, wl[...], preferred_element_type=jnp.float32)
                  + bl[...]
                  + jnp.dot(h[...], wr[...],
                            preferred_element_type=jnp.float32))


def _dense2(p, rcn, h, Wl, bl, Wr):
    return pl.pallas_call(
        _dense2_body,
        grid=(GRID,),
        in_specs=[
            pl.BlockSpec((2, BLK, F), lambda i: (0, i, 0)),
            pl.BlockSpec((BLK, 1), lambda i: (i, 0)),
            pl.BlockSpec((BLK, F), lambda i: (i, 0)),
            pl.BlockSpec((F, F), lambda i: (0, 0)),
            pl.BlockSpec((1, F), lambda i: (0, 0)),
            pl.BlockSpec((F, F), lambda i: (0, 0)),
        ],
        out_specs=pl.BlockSpec((BLK, F), lambda i: (i, 0)),
        out_shape=jax.ShapeDtypeStruct((N, F), jnp.float32),
    )(p, rcn, h, Wl.T, bl.reshape(1, F), Wr.T)


def _dense3_body(p, rcn, h, wl, bl, wr, o_ref):
    agg = jnp.sum(p[...], axis=0) * rcn[...]
    o_ref[...] = (jnp.dot(agg, wl[...], preferred_element_type=jnp.float32)
                  + bl[...]
                  + jnp.dot(h[...], wr[...],
                            preferred_element_type=jnp.float32))


def _dense3(p, rcn, h, Wl, bl, Wr):
    return pl.pallas_call(
        _dense3_body,
        out_shape=jax.ShapeDtypeStruct((OPAD, 3), jnp.float32),
    )(p, rcn, h, Wl.T, bl.reshape(1, 3), Wr.T)


def kernel(x, edge_index, pos_W, pos_b,
           Wl1, bl1, Wr1, Wl2, bl2, Wr2, Wl3, bl3, Wr3):
    src = edge_index[0].reshape(NCH, JSUB, SUB)
    dst = edge_index[1].reshape(NCH, JSUB, SUB)
    zeros_nf = jnp.zeros((N, F), jnp.float32)

    h0ext = _prep(x, pos_W, pos_b)
    p = _agg(src, dst, h0ext, zeros_nf, F)
    h1, rcn = _dense1(p, h0ext, Wl1, bl1, Wr1)
    p2 = _agg(src, dst, h1, zeros_nf, F)
    h2 = _dense2(p2, rcn, h1, Wl2, bl2, Wr2)
    p3 = _agg68(src, dst, h2)
    out = _dense3(p3, rcn[:OPAD], h2[:OPAD], Wl3, bl3, Wr3)
    return out[:NOUT]
